# baseline (speedup 1.0000x reference)
import functools

import jax
import jax.numpy as jnp
from jax import lax
from jax.experimental import pallas as pl
from jax.experimental.pallas import tpu as pltpu
from jax.experimental.pallas import tpu_sc as plsc

R = 16384
C = 4096
G = 128
L = 16
NC = 2
NS = 16
NW = NC * NS
ROWS_PER_W = R // NW
K = 16
CHUNKS = ROWS_PER_W // K


def _body(x_hbm, inds_hbm, out_hbm, inds_v, idx_v, buf_v, sem):
    wid = lax.axis_index("s") * NC + lax.axis_index("c")
    row0 = wid * ROWS_PER_W

    pltpu.sync_copy(inds_hbm, inds_v)
    ivecs = [inds_v[pl.ds(t * L, L)] for t in range(G // L)]

    def chunk(c, carry):
        base_row = row0 + c * K
        for j in range(K):
            off = (base_row + j) * C
            for t in range(G // L):
                idx_v[j, pl.ds(t * L, L)] = ivecs[t] + off
        copies = [
            pltpu.async_copy(x_hbm.at[idx_v.at[j]], buf_v.at[j], sem)
            for j in range(K)
        ]
        for cp in copies:
            cp.wait()
        pltpu.sync_copy(buf_v, out_hbm.at[pl.ds(base_row, K)])
        return carry

    lax.fori_loop(0, CHUNKS, chunk, 0)


@jax.jit
def kernel(x, inds):
    x_flat = x.reshape(R * C)
    inds_flat = inds.reshape(G).astype(jnp.int32)
    mesh = plsc.VectorSubcoreMesh(core_axis_name="c", subcore_axis_name="s")
    run = functools.partial(
        pl.kernel,
        mesh=mesh,
        out_type=jax.ShapeDtypeStruct((R, G), jnp.float32),
        scratch_types=[
            pltpu.VMEM((G,), jnp.int32),
            pltpu.VMEM((K, G), jnp.int32),
            pltpu.VMEM((K, G), jnp.float32),
            pltpu.SemaphoreType.DMA,
        ],
    )(_body)
    return run(x_flat, inds_flat)


# speedup vs baseline: 1.0797x; 1.0797x over previous
import functools

import jax
import jax.numpy as jnp
from jax import lax
from jax.experimental import pallas as pl
from jax.experimental.pallas import tpu as pltpu
from jax.experimental.pallas import tpu_sc as plsc

R = 16384
C = 4096
G = 128
L = 16
NC = 2
NS = 16
NW = NC * NS
ROWS_PER_W = R // NW
K = 16
CHUNKS = ROWS_PER_W // K
NBUF = 4


def _body(x_hbm, inds_hbm, out_hbm, inds_v, idx_v, buf_v, *sems):
    wid = lax.axis_index("s") * NC + lax.axis_index("c")
    row0 = wid * ROWS_PER_W

    pltpu.sync_copy(inds_hbm, inds_v)
    ivecs = [inds_v[pl.ds(t * L, L)] for t in range(G // L)]

    def fire(c, b):
        base_row = row0 + c * K
        for j in range(K):
            off = (base_row + j) * C
            for t in range(G // L):
                idx_v[b, j, pl.ds(t * L, L)] = ivecs[t] + off
        for j in range(K):
            pltpu.async_copy(
                x_hbm.at[idx_v.at[b].at[j]], buf_v.at[b].at[j], sems[b]
            )

    def drain(c, b):
        base_row = row0 + c * K
        pltpu.make_async_copy(
            out_hbm.at[pl.ds(base_row, K)], buf_v.at[b], sems[b]
        ).wait()
        pltpu.sync_copy(buf_v.at[b], out_hbm.at[pl.ds(base_row, K)])

    for b in range(NBUF):
        fire(b, b)

    def group(g, carry):
        for b in range(NBUF):
            drain((g - 1) * NBUF + b, b)
            fire(g * NBUF + b, b)
        return carry

    lax.fori_loop(1, CHUNKS // NBUF, group, 0)

    for b in range(NBUF):
        drain(CHUNKS - NBUF + b, b)


@jax.jit
def kernel(x, inds):
    x_flat = x.reshape(R * C)
    inds_flat = inds.reshape(G).astype(jnp.int32)
    mesh = plsc.VectorSubcoreMesh(core_axis_name="c", subcore_axis_name="s")
    run = functools.partial(
        pl.kernel,
        mesh=mesh,
        out_type=jax.ShapeDtypeStruct((R, G), jnp.float32),
        scratch_types=[
            pltpu.VMEM((G,), jnp.int32),
            pltpu.VMEM((NBUF, K, G), jnp.int32),
            pltpu.VMEM((NBUF, K, G), jnp.float32),
        ] + [pltpu.SemaphoreType.DMA] * NBUF,
    )(_body)
    return run(x_flat, inds_flat)
